# Initial kernel scaffold; baseline (speedup 1.0000x reference)
import jax
import jax.numpy as jnp
from jax.experimental import pallas as pl


def kernel(x, edge_index, edge_attr, batch, W_proj, b_proj, W_e1, b_e1, W_e2, b_e2, W_root, b_conv, W_gru_ih, b_gru_ih, W_gru_hh, b_gru_hh, W_r1, b_r1, W_r2, b_r2, W_p, b_p):
    raise NotImplementedError("write your pallas kernel here")


# speedup vs baseline: 3.4538x; 3.4538x over previous
import functools
import jax
import jax.numpy as jnp
from jax import lax
from jax.experimental import pallas as pl
from jax.experimental.pallas import tpu as pltpu
from jax.experimental.pallas import tpu_sc as plsc

_N = 10000
_E = 320000
_H = 8
_DF = 128
_DE = 16
_NG = 64
_STEPS = 3

_NC = 2
_NS = 16
_EDGES_PER_SC = _E // _NC
_EDGES_PER_TILE = _EDGES_PER_SC // _NS
_B = 400
_NB = _EDGES_PER_TILE // _B
_CH = 100
_NCH = _B // _CH
_RSTRIDE = 624
_RPT = 640


def _proj_body(x_ref, w_ref, b_ref, o_ref):
    o_ref[...] = jnp.maximum(
        jnp.dot(x_ref[...], w_ref[...], preferred_element_type=jnp.float32)
        + b_ref[...], 0.0)


def _proj(x, w, b):
    return pl.pallas_call(
        _proj_body,
        out_shape=jax.ShapeDtypeStruct((_N, _H), jnp.float32),
    )(x, w, b.reshape(1, _H))


_EB = 10000


def _edge_mlp_body(ea_ref, w1_ref, b1_ref, w2_ref, b2_ref, o_ref):
    eh = jnp.maximum(
        jnp.dot(ea_ref[...], w1_ref[...], preferred_element_type=jnp.float32)
        + b1_ref[...], 0.0)
    o_ref[...] = (
        jnp.dot(eh, w2_ref[...], preferred_element_type=jnp.float32)
        + b2_ref[...])


def _edge_mlp(edge_attr, w1, b1, w2, b2):
    grid = _E // _EB
    return pl.pallas_call(
        _edge_mlp_body,
        grid=(grid,),
        in_specs=[
            pl.BlockSpec((_EB, _DE), lambda i: (i, 0)),
            pl.BlockSpec((_DE, 16), lambda i: (0, 0)),
            pl.BlockSpec((1, 16), lambda i: (0, 0)),
            pl.BlockSpec((16, _H * _H), lambda i: (0, 0)),
            pl.BlockSpec((1, _H * _H), lambda i: (0, 0)),
        ],
        out_specs=pl.BlockSpec((_EB, _H * _H), lambda i: (i, 0)),
        out_shape=jax.ShapeDtypeStruct((_E, _H * _H), jnp.float32),
    )(edge_attr, w1, b1.reshape(1, 16), w2, b2.reshape(1, _H * _H))


def _sigmoid(v):
    return 1.0 / (1.0 + jnp.exp(-v))


def _gru_body(aggp_ref, h_ref, wr_ref, bc_ref, wih_ref, bih_ref,
              whh_ref, bhh_ref, o_ref):
    agg = aggp_ref[pl.ds(0, _N), :] + aggp_ref[pl.ds(_N, _N), :]
    h = h_ref[...]
    m = jnp.maximum(
        agg + jnp.dot(h, wr_ref[...], preferred_element_type=jnp.float32)
        + bc_ref[...], 0.0)
    gi = jnp.dot(m, wih_ref[...], preferred_element_type=jnp.float32) + bih_ref[...]
    gh = jnp.dot(h, whh_ref[...], preferred_element_type=jnp.float32) + bhh_ref[...]
    r = _sigmoid(gi[:, 0:_H] + gh[:, 0:_H])
    z = _sigmoid(gi[:, _H:2 * _H] + gh[:, _H:2 * _H])
    n = jnp.tanh(gi[:, 2 * _H:3 * _H] + r * gh[:, 2 * _H:3 * _H])
    o_ref[...] = (1.0 - z) * n + z * h


def _gru(aggp, h, wr, bc, wih, bih, whh, bhh):
    return pl.pallas_call(
        _gru_body,
        out_shape=jax.ShapeDtypeStruct((_N, _H), jnp.float32),
    )(aggp, h, wr, bc.reshape(1, _H), wih, bih.reshape(1, 3 * _H),
      whh, bhh.reshape(1, 3 * _H))


def _readout_body(h_ref, batch_ref, w1_ref, b1_ref, w2_ref, b2_ref,
                  wp_ref, bp_ref, o_ref):
    h = h_ref[...]
    nf = jnp.maximum(
        jnp.dot(h, w1_ref[...], preferred_element_type=jnp.float32)
        + b1_ref[...], 0.0)
    nf = jnp.dot(nf, w2_ref[...], preferred_element_type=jnp.float32) + b2_ref[...]
    gids = lax.broadcasted_iota(jnp.int32, (_N, _NG), 1)
    oh = (batch_ref[...] == gids).astype(jnp.float32)
    sums = lax.dot_general(oh, nf, (((0,), (0,)), ((), ())),
                           preferred_element_type=jnp.float32)
    counts = jnp.sum(oh, axis=0).reshape(_NG, 1)
    g = sums / jnp.maximum(counts, 1.0)
    o_ref[...] = (
        jnp.dot(g, wp_ref[...], preferred_element_type=jnp.float32) + bp_ref[...])


def _readout(h, batch2d, w1, b1, w2, b2, wp, bp):
    return pl.pallas_call(
        _readout_body,
        out_shape=jax.ShapeDtypeStruct((_NG, 1), jnp.float32),
    )(h, batch2d, w1, b1.reshape(1, _H), w2, b2.reshape(1, _H),
      wp, bp.reshape(1, 1))


def _sc_step_body(h_hbm, src_hbm, dst_hbm, ew_hbm, zeros_hbm, out_hbm,
                  h_buf, ew_buf, msg_buf, sidx_buf, didx_buf, stage_buf,
                  agg_sh):
    c = lax.axis_index("c")
    s = lax.axis_index("s")
    r0 = s * _RSTRIDE

    pltpu.sync_copy(h_hbm, h_buf)
    pltpu.sync_copy(zeros_hbm, stage_buf)
    pltpu.sync_copy(stage_buf, agg_sh.at[pl.ds(r0, _RPT)])
    plsc.subcore_barrier()

    lanes = lax.iota(jnp.int32, 16)
    tile_base = c * _EDGES_PER_SC + s * _EDGES_PER_TILE
    tile_crow = c * (_EDGES_PER_SC // _CH) + s * (_EDGES_PER_TILE // _CH)

    def blk_body(blk, carry):
        e0 = tile_base + blk * _B
        crow0 = tile_crow + blk * _NCH
        pltpu.sync_copy(src_hbm.at[pl.ds(e0, _B)], sidx_buf)
        pltpu.sync_copy(dst_hbm.at[pl.ds(crow0, _NCH)], didx_buf)
        pltpu.sync_copy(ew_hbm.at[pl.ds(e0, _B)], ew_buf)

        def grp_body(g, carry2):
            srows = sidx_buf[pl.ds(g * 16, 16)]
            mrows = lanes + g * 16
            hc = [plsc.load_gather(h_buf, [srows, jnp.full((16,), i, jnp.int32)])
                  for i in range(_H)]
            for o in range(_H):
                acc = hc[0] * plsc.load_gather(
                    ew_buf, [mrows, jnp.full((16,), o, jnp.int32)])
                for i in range(1, _H):
                    acc = acc + hc[i] * plsc.load_gather(
                        ew_buf, [mrows, jnp.full((16,), i * _H + o, jnp.int32)])
                plsc.store_scatter(msg_buf, [mrows, jnp.full((16,), o, jnp.int32)],
                                   acc)
            return carry2

        lax.fori_loop(0, _B // 16, grp_body, 0)

        for ch in range(_NCH):
            pltpu.sync_copy(msg_buf.at[pl.ds(ch * _CH, _CH)],
                            agg_sh.at[didx_buf.at[ch]], add=True)
        return carry

    lax.fori_loop(0, _NB, blk_body, 0)
    plsc.subcore_barrier()

    pltpu.sync_copy(agg_sh.at[pl.ds(r0, _RPT)], stage_buf)
    pltpu.sync_copy(stage_buf, out_hbm.at[pl.ds(c * _N + r0, _RPT)])


@functools.cache
def _make_sc_step():
    return pl.kernel(
        _sc_step_body,
        out_type=jax.ShapeDtypeStruct((2 * _N, _H), jnp.float32),
        mesh=plsc.VectorSubcoreMesh(core_axis_name="c", subcore_axis_name="s"),
        compiler_params=pltpu.CompilerParams(
            needs_layout_passes=False, use_tc_tiling_on_sc=False),
        scratch_types=[
            pltpu.VMEM((_N, _H), jnp.float32),
            pltpu.VMEM((_B, _H * _H), jnp.float32),
            pltpu.VMEM((_B, _H), jnp.float32),
            pltpu.VMEM((_B,), jnp.int32),
            pltpu.VMEM((_NCH, _CH), jnp.int32),
            pltpu.VMEM((_RPT, _H), jnp.float32),
            pltpu.VMEM_SHARED((_N, _H), jnp.float32),
        ],
    )


def _sc_step(h, src, dst2d, ew, zeros):
    return _make_sc_step()(h, src, dst2d, ew, zeros)


@jax.jit
def kernel(x, edge_index, edge_attr, batch, W_proj, b_proj, W_e1, b_e1,
           W_e2, b_e2, W_root, b_conv, W_gru_ih, b_gru_ih, W_gru_hh,
           b_gru_hh, W_r1, b_r1, W_r2, b_r2, W_p, b_p):
    src = edge_index[0].astype(jnp.int32)
    dst2d = edge_index[1].astype(jnp.int32).reshape(_E // _CH, _CH)
    batch2d = batch.astype(jnp.int32).reshape(_N, 1)
    zeros = jnp.zeros((_RPT, _H), jnp.float32)

    h = _proj(x, W_proj, b_proj)
    ew = _edge_mlp(edge_attr, W_e1, b_e1, W_e2, b_e2)

    for _ in range(_STEPS):
        aggp = _sc_step(h, src, dst2d, ew, zeros)
        h = _gru(aggp, h, W_root, b_conv, W_gru_ih, b_gru_ih,
                 W_gru_hh, b_gru_hh)

    return _readout(h, batch2d, W_r1, b_r1, W_r2, b_r2, W_p, b_p)
